# Initial kernel scaffold; baseline (speedup 1.0000x reference)
import jax
import jax.numpy as jnp
from jax.experimental import pallas as pl


def kernel(data, edge, edge_feature, emb, W_msg0, b_msg0, W_edge0, b_edge0, gamma0, beta0, W_msg1, b_msg1, W_edge1, b_edge1, gamma1, beta1):
    raise NotImplementedError("write your pallas kernel here")


# speedup vs baseline: 4.3112x; 4.3112x over previous
import functools

import jax
import jax.numpy as jnp
from jax import lax
from jax.experimental import pallas as pl
from jax.experimental.pallas import tpu as pltpu
from jax.experimental.pallas import tpu_sc as plsc

N = 10000
E = 320000
D = 128
DEPTH_SIZE = 64
EPS = 1e-5

NC = 2
NS = 16
NW = NC * NS
EPT = E // NW
K = 80
NCHUNK = EPT // K
RCH = 80
NRCH = N // RCH
RITER = -(-NRCH // NS)

_mesh = plsc.VectorSubcoreMesh(core_axis_name="c", subcore_axis_name="s")


def _sc_agg_body(tab_hbm, idx_hbm, agg_hbm, zx_hbm, px_hbm,
                 idx_v, agg_v, rows_v, acc_sh, sem):
    c = lax.axis_index("c")
    s = lax.axis_index("s")

    def rows_foreach(fn):
        for i in range(RITER):
            rc = s + NS * i

            @pl.when(rc < NRCH)
            def _():
                fn(rc * RCH)

    rows_foreach(lambda r0: pltpu.sync_copy(zx_hbm.at[pl.ds(r0, RCH)],
                                            acc_sh.at[pl.ds(r0, RCH)]))
    plsc.subcore_barrier()

    base = (c * NS + s) * EPT

    def step(i, carry):
        off = base + i * K
        pltpu.sync_copy(idx_hbm.at[pl.ds(off, K)], idx_v)
        pltpu.sync_copy(agg_hbm.at[pl.ds(off, K)], agg_v)
        pltpu.async_copy(tab_hbm.at[idx_v], rows_v, sem).wait()
        pltpu.sync_copy(rows_v, acc_sh.at[agg_v], add=True)
        return carry

    lax.fori_loop(0, NCHUNK, step, 0)
    plsc.subcore_barrier()
    rows_foreach(lambda r0: pltpu.sync_copy(acc_sh.at[pl.ds(r0, RCH)],
                                            px_hbm.at[c, pl.ds(r0, RCH)]))


def _make_sc_agg(tab_rows):
    return pl.kernel(
        _sc_agg_body,
        mesh=_mesh,
        out_type=[jax.ShapeDtypeStruct((NC, N, D), jnp.float32)],
        scratch_types=[
            pltpu.VMEM((K,), jnp.int32),
            pltpu.VMEM((K,), jnp.int32),
            pltpu.VMEM((K, D), jnp.float32),
            pltpu.VMEM_SHARED((N, D), jnp.float32),
            pltpu.SemaphoreType.DMA,
        ],
    )


_sc_agg_x = _make_sc_agg(N)
_sc_agg_cnt = _make_sc_agg(DEPTH_SIZE)


def _dense_body(px_ref, pc_ref, x_ref, wm_ref, wc_ref, g_ref, b_ref, o_ref):
    ax = px_ref[0] + px_ref[1]
    cnt = pc_ref[0] + pc_ref[1]
    h = jnp.dot(ax, wm_ref[...], preferred_element_type=jnp.float32)
    h = h + jnp.dot(cnt, wc_ref[...], preferred_element_type=jnp.float32)
    h = h + x_ref[...]
    mean = jnp.mean(h, axis=0, keepdims=True)
    ctr = h - mean
    var = jnp.mean(ctr * ctr, axis=0, keepdims=True)
    o = g_ref[...] * ctr * lax.rsqrt(var + EPS) + b_ref[...]
    o_ref[...] = jnp.maximum(o, 0.0)


_dense = pl.pallas_call(
    _dense_body,
    out_shape=jax.ShapeDtypeStruct((N, D), jnp.float32),
)


def kernel(data, edge, edge_feature, emb,
           W_msg0, b_msg0, W_edge0, b_edge0, gamma0, beta0,
           W_msg1, b_msg1, W_edge1, b_edge1, gamma1, beta1):
    agg = edge[0]
    src = edge[1]
    onehot = jnp.eye(DEPTH_SIZE, D, dtype=jnp.float32)
    zx = jnp.zeros((N, D), jnp.float32)

    (pc,) = _sc_agg_cnt(onehot, edge_feature, agg, zx)
    (px0,) = _sc_agg_x(data, src, agg, zx)

    def edge_mat(W_edge, b_msg, b_edge):
        m = jnp.dot(emb, W_edge) + (b_msg + b_edge)[None, :]
        return jnp.zeros((D, D), jnp.float32).at[:DEPTH_SIZE].set(m)

    x1 = _dense(px0, pc, data, W_msg0, edge_mat(W_edge0, b_msg0, b_edge0),
                gamma0.reshape(1, D), beta0.reshape(1, D))

    (px1,) = _sc_agg_x(x1, src, agg, zx)

    out = _dense(px1, pc, x1, W_msg1, edge_mat(W_edge1, b_msg1, b_edge1),
                 gamma1.reshape(1, D), beta1.reshape(1, D))
    return out


# speedup vs baseline: 6.2493x; 1.4496x over previous
import functools

import jax
import jax.numpy as jnp
from jax import lax
from jax.experimental import pallas as pl
from jax.experimental.pallas import tpu as pltpu
from jax.experimental.pallas import tpu_sc as plsc

N = 10000
E = 320000
D = 128
DEPTH_SIZE = 64
EPS = 1e-5

NC = 2
NS = 16
NW = NC * NS
EPT = E // NW
K = 80
NCHUNK = EPT // K
NBUF = 2
GITER = -(-NCHUNK // NBUF)
RCH = 80
NRCH = N // RCH
RITER = -(-NRCH // NS)

_mesh = plsc.VectorSubcoreMesh(core_axis_name="c", subcore_axis_name="s")


def _sc_agg_body(tab_hbm, idx_hbm, agg_hbm, zx_hbm, px_hbm,
                 idx_all, agg_ring, *rest):
    bufs = rest[0:NBUF]
    gsem = rest[NBUF:2 * NBUF]
    ssem = rest[2 * NBUF:3 * NBUF]
    asem = rest[3 * NBUF:4 * NBUF]
    acc_sh = rest[4 * NBUF]
    c = lax.axis_index("c")
    s = lax.axis_index("s")

    def rows_foreach(fn):
        for i in range(RITER):
            rc = s + NS * i

            @pl.when(rc < NRCH)
            def _():
                fn(rc * RCH)

    rows_foreach(lambda r0: pltpu.sync_copy(zx_hbm.at[pl.ds(r0, RCH)],
                                            acc_sh.at[pl.ds(r0, RCH)]))

    wid = c * NS + s
    pltpu.sync_copy(idx_hbm.at[wid], idx_all)
    plsc.subcore_barrier()

    def gstart(j, b):
        pltpu.async_copy(tab_hbm.at[idx_all.at[j]], bufs[b], gsem[b])

    def gwait(j, b):
        pltpu.make_async_copy(tab_hbm.at[idx_all.at[j]], bufs[b], gsem[b]).wait()

    def astart(j, b):
        pltpu.async_copy(agg_hbm.at[wid, j], agg_ring.at[b], asem[b])

    def await_(j, b):
        pltpu.make_async_copy(agg_hbm.at[wid, j], agg_ring.at[b], asem[b]).wait()

    def sstart(j, b):
        pltpu.async_copy(bufs[b], acc_sh.at[agg_ring.at[b]], ssem[b], add=True)

    def swait(j, b):
        pltpu.make_async_copy(bufs[b], acc_sh.at[agg_ring.at[b]], ssem[b]).wait()

    astart(0, 0)
    gstart(0, 0)

    def group(g, carry):
        for t in range(NBUF):
            i = NBUF * g + t

            @pl.when((i >= 1) & (i < NCHUNK))
            def _():
                swait(i - 1, 1 - t)

            @pl.when(i + 1 < NCHUNK)
            def _():
                astart(i + 1, 1 - t)
                gstart(i + 1, 1 - t)

            @pl.when(i < NCHUNK)
            def _():
                gwait(i, t)
                await_(i, t)
                sstart(i, t)
        return carry

    lax.fori_loop(0, GITER, group, 0)
    swait(NCHUNK - 1, (NCHUNK - 1) % NBUF)
    plsc.subcore_barrier()

    def write_chunk(r0):
        pltpu.sync_copy(acc_sh.at[pl.ds(r0, RCH)], px_hbm.at[c, pl.ds(r0, RCH)])

    rows_foreach(write_chunk)


_sc_agg = pl.kernel(
    _sc_agg_body,
    mesh=_mesh,
    out_type=[jax.ShapeDtypeStruct((NC, N, D), jnp.float32)],
    scratch_types=(
        [pltpu.VMEM((NCHUNK, K), jnp.int32),
         pltpu.VMEM((NBUF, K), jnp.int32)]
        + [pltpu.VMEM((K, D), jnp.float32)] * NBUF
        + [pltpu.SemaphoreType.DMA] * (3 * NBUF)
        + [pltpu.VMEM_SHARED((N, D), jnp.float32)]
    ),
)


def _dense_body(px_ref, pc_ref, x_ref, wm_ref, wc_ref, g_ref, b_ref, o_ref):
    ax = px_ref[0] + px_ref[1]
    cnt = pc_ref[0] + pc_ref[1]
    h = jnp.dot(ax, wm_ref[...], preferred_element_type=jnp.float32)
    h = h + jnp.dot(cnt, wc_ref[...], preferred_element_type=jnp.float32)
    h = h + x_ref[...]
    mean = jnp.mean(h, axis=0, keepdims=True)
    ctr = h - mean
    var = jnp.mean(ctr * ctr, axis=0, keepdims=True)
    o = g_ref[...] * ctr * lax.rsqrt(var + EPS) + b_ref[...]
    o_ref[...] = jnp.maximum(o, 0.0)


_dense = pl.pallas_call(
    _dense_body,
    out_shape=jax.ShapeDtypeStruct((N, D), jnp.float32),
)


def kernel(data, edge, edge_feature, emb,
           W_msg0, b_msg0, W_edge0, b_edge0, gamma0, beta0,
           W_msg1, b_msg1, W_edge1, b_edge1, gamma1, beta1):
    agg = edge[0].reshape(NW, NCHUNK, K)
    src = edge[1].reshape(NW, NCHUNK, K)
    feat = edge_feature.reshape(NW, NCHUNK, K)
    onehot = jnp.eye(DEPTH_SIZE, D, dtype=jnp.float32)
    zx = jnp.zeros((N, D), jnp.float32)

    (pc,) = _sc_agg(onehot, feat, agg, zx)
    (px0,) = _sc_agg(data, src, agg, zx)

    def edge_mat(W_edge, b_msg, b_edge):
        m = jnp.dot(emb, W_edge) + (b_msg + b_edge)[None, :]
        return jnp.zeros((D, D), jnp.float32).at[:DEPTH_SIZE].set(m)

    x1 = _dense(px0, pc, data, W_msg0, edge_mat(W_edge0, b_msg0, b_edge0),
                gamma0.reshape(1, D), beta0.reshape(1, D))

    (px1,) = _sc_agg(x1, src, agg, zx)

    out = _dense(px1, pc, x1, W_msg1, edge_mat(W_edge1, b_msg1, b_edge1),
                 gamma1.reshape(1, D), beta1.reshape(1, D))
    return out


# speedup vs baseline: 6.5883x; 1.0543x over previous
import functools

import jax
import jax.numpy as jnp
from jax import lax
from jax.experimental import pallas as pl
from jax.experimental.pallas import tpu as pltpu
from jax.experimental.pallas import tpu_sc as plsc

N = 10000
E = 320000
D = 128
DEPTH_SIZE = 64
EPS = 1e-5

NC = 2
NS = 16
NW = NC * NS
EPT = E // NW
K = 80
NCHUNK = EPT // K
NBUF = 3
GITER = -(-NCHUNK // NBUF)
RCH = 80
NRCH = N // RCH
RITER = -(-NRCH // NS)

_mesh = plsc.VectorSubcoreMesh(core_axis_name="c", subcore_axis_name="s")


def _sc_agg_body(tab_hbm, idx_hbm, agg_hbm, zx_hbm, px_hbm,
                 idx_all, agg_ring, *rest):
    bufs = rest[0:NBUF]
    gsem = rest[NBUF:2 * NBUF]
    ssem = rest[2 * NBUF:3 * NBUF]
    asem = rest[3 * NBUF:4 * NBUF]
    acc_sh = rest[4 * NBUF]
    c = lax.axis_index("c")
    s = lax.axis_index("s")

    def rows_foreach(fn):
        for i in range(RITER):
            rc = s + NS * i

            @pl.when(rc < NRCH)
            def _():
                fn(rc * RCH)

    rows_foreach(lambda r0: pltpu.sync_copy(zx_hbm.at[pl.ds(r0, RCH)],
                                            acc_sh.at[pl.ds(r0, RCH)]))

    wid = c * NS + s
    pltpu.sync_copy(idx_hbm.at[wid], idx_all)
    plsc.subcore_barrier()

    def gstart(j, b):
        pltpu.async_copy(tab_hbm.at[idx_all.at[j]], bufs[b], gsem[b])

    def gwait(j, b):
        pltpu.make_async_copy(tab_hbm.at[idx_all.at[j]], bufs[b], gsem[b]).wait()

    def astart(j, b):
        pltpu.async_copy(agg_hbm.at[wid, j], agg_ring.at[b], asem[b])

    def await_(j, b):
        pltpu.make_async_copy(agg_hbm.at[wid, j], agg_ring.at[b], asem[b]).wait()

    def sstart(j, b):
        pltpu.async_copy(bufs[b], acc_sh.at[agg_ring.at[b]], ssem[b], add=True)

    def swait(j, b):
        pltpu.make_async_copy(bufs[b], acc_sh.at[agg_ring.at[b]], ssem[b]).wait()

    astart(0, 0)
    gstart(0, 0)

    def group(g, carry):
        for t in range(NBUF):
            i = NBUF * g + t
            tn = (t + 1) % NBUF

            @pl.when((i >= NBUF - 1) & (i < NCHUNK))
            def _():
                swait(i - (NBUF - 1), tn)

            @pl.when(i + 1 < NCHUNK)
            def _():
                astart(i + 1, tn)
                gstart(i + 1, tn)

            @pl.when(i < NCHUNK)
            def _():
                gwait(i, t)
                await_(i, t)
                sstart(i, t)
        return carry

    lax.fori_loop(0, GITER, group, 0)
    for j in range(NCHUNK - (NBUF - 1), NCHUNK):
        swait(j, j % NBUF)
    plsc.subcore_barrier()

    def write_chunk(r0):
        pltpu.sync_copy(acc_sh.at[pl.ds(r0, RCH)], px_hbm.at[c, pl.ds(r0, RCH)])

    rows_foreach(write_chunk)


_sc_agg = pl.kernel(
    _sc_agg_body,
    mesh=_mesh,
    out_type=[jax.ShapeDtypeStruct((NC, N, D), jnp.float32)],
    scratch_types=(
        [pltpu.VMEM((NCHUNK, K), jnp.int32),
         pltpu.VMEM((NBUF, K), jnp.int32)]
        + [pltpu.VMEM((K, D), jnp.float32)] * NBUF
        + [pltpu.SemaphoreType.DMA] * (3 * NBUF)
        + [pltpu.VMEM_SHARED((N, D), jnp.float32)]
    ),
)


def _dense_body(px_ref, pc_ref, x_ref, wm_ref, wc_ref, g_ref, b_ref, o_ref):
    ax = px_ref[0] + px_ref[1]
    cnt = pc_ref[0] + pc_ref[1]
    h = jnp.dot(ax, wm_ref[...], preferred_element_type=jnp.float32)
    h = h + jnp.dot(cnt, wc_ref[...], preferred_element_type=jnp.float32)
    h = h + x_ref[...]
    mean = jnp.mean(h, axis=0, keepdims=True)
    ctr = h - mean
    var = jnp.mean(ctr * ctr, axis=0, keepdims=True)
    o = g_ref[...] * ctr * lax.rsqrt(var + EPS) + b_ref[...]
    o_ref[...] = jnp.maximum(o, 0.0)


_dense = pl.pallas_call(
    _dense_body,
    out_shape=jax.ShapeDtypeStruct((N, D), jnp.float32),
)


def kernel(data, edge, edge_feature, emb,
           W_msg0, b_msg0, W_edge0, b_edge0, gamma0, beta0,
           W_msg1, b_msg1, W_edge1, b_edge1, gamma1, beta1):
    agg = edge[0].reshape(NW, NCHUNK, K)
    src = edge[1].reshape(NW, NCHUNK, K)
    feat = edge_feature.reshape(NW, NCHUNK, K)
    onehot = jnp.eye(DEPTH_SIZE, D, dtype=jnp.float32)
    zx = jnp.zeros((N, D), jnp.float32)

    (pc,) = _sc_agg(onehot, feat, agg, zx)
    (px0,) = _sc_agg(data, src, agg, zx)

    def edge_mat(W_edge, b_msg, b_edge):
        m = jnp.dot(emb, W_edge) + (b_msg + b_edge)[None, :]
        return jnp.zeros((D, D), jnp.float32).at[:DEPTH_SIZE].set(m)

    x1 = _dense(px0, pc, data, W_msg0, edge_mat(W_edge0, b_msg0, b_edge0),
                gamma0.reshape(1, D), beta0.reshape(1, D))

    (px1,) = _sc_agg(x1, src, agg, zx)

    out = _dense(px1, pc, x1, W_msg1, edge_mat(W_edge1, b_msg1, b_edge1),
                 gamma1.reshape(1, D), beta1.reshape(1, D))
    return out


# speedup vs baseline: 11.4600x; 1.7394x over previous
import functools

import jax
import jax.numpy as jnp
from jax import lax
from jax.experimental import pallas as pl
from jax.experimental.pallas import tpu as pltpu
from jax.experimental.pallas import tpu_sc as plsc

N = 10000
E = 320000
D = 128
DEPTH_SIZE = 64
EPS = 1e-5

NC = 2
NS = 16
NW = NC * NS
EPT = E // NW
K = 80
NCHUNK = EPT // K
NBUF = 3
GITER = -(-NCHUNK // NBUF)
RCH = 80
NRCH = N // RCH
RITER = -(-NRCH // NS)

_mesh = plsc.VectorSubcoreMesh(core_axis_name="c", subcore_axis_name="s")


def _sc_agg_body(tab_hbm, idx_hbm, agg_hbm, zx_hbm, px_hbm,
                 idx_all, agg_ring, *rest):
    bufs = rest[0:NBUF]
    gsem = rest[NBUF:2 * NBUF]
    ssem = rest[2 * NBUF:3 * NBUF]
    asem = rest[3 * NBUF:4 * NBUF]
    acc_sh = rest[4 * NBUF]
    c = lax.axis_index("c")
    s = lax.axis_index("s")

    def rows_foreach(fn):
        for i in range(RITER):
            rc = s + NS * i

            @pl.when(rc < NRCH)
            def _():
                fn(rc * RCH)

    rows_foreach(lambda r0: pltpu.sync_copy(zx_hbm.at[pl.ds(r0, RCH)],
                                            acc_sh.at[pl.ds(r0, RCH)]))

    wid = c * NS + s
    pltpu.sync_copy(idx_hbm.at[wid], idx_all)
    plsc.subcore_barrier()

    def gstart(j, b):
        pltpu.async_copy(tab_hbm.at[idx_all.at[j]], bufs[b], gsem[b])

    def gwait(j, b):
        pltpu.make_async_copy(tab_hbm.at[idx_all.at[j]], bufs[b], gsem[b]).wait()

    def astart(j, b):
        pltpu.async_copy(agg_hbm.at[wid, j], agg_ring.at[b], asem[b])

    def await_(j, b):
        pltpu.make_async_copy(agg_hbm.at[wid, j], agg_ring.at[b], asem[b]).wait()

    def sstart(j, b):
        pltpu.async_copy(bufs[b], acc_sh.at[agg_ring.at[b]], ssem[b], add=True)

    def swait(j, b):
        pltpu.make_async_copy(bufs[b], acc_sh.at[agg_ring.at[b]], ssem[b]).wait()

    astart(0, 0)
    gstart(0, 0)

    def group(g, carry):
        for t in range(NBUF):
            i = NBUF * g + t
            tn = (t + 1) % NBUF

            @pl.when((i >= NBUF - 1) & (i < NCHUNK))
            def _():
                swait(i - (NBUF - 1), tn)

            @pl.when(i + 1 < NCHUNK)
            def _():
                astart(i + 1, tn)
                gstart(i + 1, tn)

            @pl.when(i < NCHUNK)
            def _():
                gwait(i, t)
                await_(i, t)
                sstart(i, t)
        return carry

    lax.fori_loop(0, GITER, group, 0)
    for j in range(NCHUNK - (NBUF - 1), NCHUNK):
        swait(j, j % NBUF)
    plsc.subcore_barrier()

    def write_chunk(r0):
        pltpu.sync_copy(acc_sh.at[pl.ds(r0, RCH)], px_hbm.at[c, pl.ds(r0, RCH)])

    rows_foreach(write_chunk)


_sc_agg = pl.kernel(
    _sc_agg_body,
    mesh=_mesh,
    out_type=[jax.ShapeDtypeStruct((NC, N, D), jnp.float32)],
    scratch_types=(
        [pltpu.VMEM((NCHUNK, K), jnp.int32),
         pltpu.VMEM((NBUF, K), jnp.int32)]
        + [pltpu.VMEM((K, D), jnp.float32)] * NBUF
        + [pltpu.SemaphoreType.DMA] * (3 * NBUF)
        + [pltpu.VMEM_SHARED((N, D), jnp.float32)]
    ),
)


def _dense_body(px_ref, pc_ref, x_ref, wm_ref, wc_ref, g_ref, b_ref, o_ref):
    ax = px_ref[0] + px_ref[1]
    cnt = pc_ref[0] + pc_ref[1]
    h = jnp.dot(ax, wm_ref[...], preferred_element_type=jnp.float32)
    h = h + jnp.dot(cnt, wc_ref[...], preferred_element_type=jnp.float32)
    h = h + x_ref[...]
    mean = jnp.mean(h, axis=0, keepdims=True)
    ctr = h - mean
    var = jnp.mean(ctr * ctr, axis=0, keepdims=True)
    o = g_ref[...] * ctr * lax.rsqrt(var + EPS) + b_ref[...]
    o_ref[...] = jnp.maximum(o, 0.0)


_dense = pl.pallas_call(
    _dense_body,
    out_shape=jax.ShapeDtypeStruct((N, D), jnp.float32),
)


def kernel(data, edge, edge_feature, emb,
           W_msg0, b_msg0, W_edge0, b_edge0, gamma0, beta0,
           W_msg1, b_msg1, W_edge1, b_edge1, gamma1, beta1):
    agg = edge[0].reshape(NW, NCHUNK, K)
    src = edge[1].reshape(NW, NCHUNK, K)
    REP = 32
    feat = (edge_feature + DEPTH_SIZE * (jnp.arange(E, dtype=jnp.int32) % REP)
            ).reshape(NW, NCHUNK, K)
    onehot = jnp.tile(jnp.eye(DEPTH_SIZE, D, dtype=jnp.float32), (REP, 1))
    zx = jnp.zeros((N, D), jnp.float32)

    (pc,) = _sc_agg(onehot, feat, agg, zx)
    (px0,) = _sc_agg(data, src, agg, zx)

    def edge_mat(W_edge, b_msg, b_edge):
        m = jnp.dot(emb, W_edge) + (b_msg + b_edge)[None, :]
        return jnp.zeros((D, D), jnp.float32).at[:DEPTH_SIZE].set(m)

    x1 = _dense(px0, pc, data, W_msg0, edge_mat(W_edge0, b_msg0, b_edge0),
                gamma0.reshape(1, D), beta0.reshape(1, D))

    (px1,) = _sc_agg(x1, src, agg, zx)

    out = _dense(px1, pc, x1, W_msg1, edge_mat(W_edge1, b_msg1, b_edge1),
                 gamma1.reshape(1, D), beta1.reshape(1, D))
    return out


# speedup vs baseline: 12.0893x; 1.0549x over previous
import functools

import jax
import jax.numpy as jnp
from jax import lax
from jax.experimental import pallas as pl
from jax.experimental.pallas import tpu as pltpu
from jax.experimental.pallas import tpu_sc as plsc

N = 10000
E = 320000
D = 128
DEPTH_SIZE = 64
EPS = 1e-5

NC = 2
NS = 16
NW = NC * NS
EPT = E // NW
K = 80
NCHUNK = EPT // K
NBUF = 3
GITER = -(-NCHUNK // NBUF)
RCH = 80
NRCH = N // RCH
RITER = -(-NRCH // NS)

_mesh = plsc.VectorSubcoreMesh(core_axis_name="c", subcore_axis_name="s")


def _sc_agg_body(tab_hbm, idx_hbm, agg_hbm, px_hbm,
                 idx_all, agg_ring, *rest):
    bufs = rest[0:NBUF]
    gsem = rest[NBUF:2 * NBUF]
    ssem = rest[2 * NBUF:3 * NBUF]
    asem = rest[3 * NBUF:4 * NBUF]
    acc_sh = rest[4 * NBUF]
    c = lax.axis_index("c")
    s = lax.axis_index("s")

    def rows_foreach(fn):
        for i in range(RITER):
            rc = s + NS * i

            @pl.when(rc < NRCH)
            def _():
                fn(rc * RCH)

    zvec = jnp.zeros((16,), jnp.float32)

    def zrow(r, carry):
        for cc in range(D // 16):
            bufs[0][r, pl.ds(cc * 16, 16)] = zvec
        return carry

    lax.fori_loop(0, K, zrow, 0)
    rows_foreach(lambda r0: pltpu.sync_copy(bufs[0],
                                            acc_sh.at[pl.ds(r0, RCH)]))

    wid = c * NS + s
    pltpu.sync_copy(idx_hbm.at[wid], idx_all)
    plsc.subcore_barrier()

    def gstart(j, b):
        pltpu.async_copy(tab_hbm.at[idx_all.at[j]], bufs[b], gsem[b])

    def gwait(j, b):
        pltpu.make_async_copy(tab_hbm.at[idx_all.at[j]], bufs[b], gsem[b]).wait()

    def astart(j, b):
        pltpu.async_copy(agg_hbm.at[wid, j], agg_ring.at[b], asem[b])

    def await_(j, b):
        pltpu.make_async_copy(agg_hbm.at[wid, j], agg_ring.at[b], asem[b]).wait()

    def sstart(j, b):
        pltpu.async_copy(bufs[b], acc_sh.at[agg_ring.at[b]], ssem[b], add=True)

    def swait(j, b):
        pltpu.make_async_copy(bufs[b], acc_sh.at[agg_ring.at[b]], ssem[b]).wait()

    astart(0, 0)
    gstart(0, 0)

    def group(g, carry):
        for t in range(NBUF):
            i = NBUF * g + t
            tn = (t + 1) % NBUF

            @pl.when((i >= NBUF - 1) & (i < NCHUNK))
            def _():
                swait(i - (NBUF - 1), tn)

            @pl.when(i + 1 < NCHUNK)
            def _():
                astart(i + 1, tn)
                gstart(i + 1, tn)

            @pl.when(i < NCHUNK)
            def _():
                gwait(i, t)
                await_(i, t)
                sstart(i, t)
        return carry

    lax.fori_loop(0, GITER, group, 0)
    for j in range(NCHUNK - (NBUF - 1), NCHUNK):
        swait(j, j % NBUF)
    plsc.subcore_barrier()

    def write_chunk(r0):
        pltpu.sync_copy(acc_sh.at[pl.ds(r0, RCH)], px_hbm.at[c, pl.ds(r0, RCH)])

    rows_foreach(write_chunk)


_sc_agg = pl.kernel(
    _sc_agg_body,
    mesh=_mesh,
    out_type=[jax.ShapeDtypeStruct((NC, N, D), jnp.float32)],
    scratch_types=(
        [pltpu.VMEM((NCHUNK, K), jnp.int32),
         pltpu.VMEM((NBUF, K), jnp.int32)]
        + [pltpu.VMEM((K, D), jnp.float32)] * NBUF
        + [pltpu.SemaphoreType.DMA] * (3 * NBUF)
        + [pltpu.VMEM_SHARED((N, D), jnp.float32)]
    ),
)


def _dense_body(px_ref, pc_ref, x_ref, wm_ref, wc_ref, g_ref, b_ref, o_ref):
    ax = px_ref[0] + px_ref[1]
    cnt = pc_ref[0] + pc_ref[1]
    h = jnp.dot(ax, wm_ref[...], preferred_element_type=jnp.float32)
    h = h + jnp.dot(cnt, wc_ref[...], preferred_element_type=jnp.float32)
    h = h + x_ref[...]
    mean = jnp.mean(h, axis=0, keepdims=True)
    ctr = h - mean
    var = jnp.mean(ctr * ctr, axis=0, keepdims=True)
    o = g_ref[...] * ctr * lax.rsqrt(var + EPS) + b_ref[...]
    o_ref[...] = jnp.maximum(o, 0.0)


_dense = pl.pallas_call(
    _dense_body,
    out_shape=jax.ShapeDtypeStruct((N, D), jnp.float32),
)


def kernel(data, edge, edge_feature, emb,
           W_msg0, b_msg0, W_edge0, b_edge0, gamma0, beta0,
           W_msg1, b_msg1, W_edge1, b_edge1, gamma1, beta1):
    agg = edge[0].reshape(NW, NCHUNK, K)
    src = edge[1].reshape(NW, NCHUNK, K)
    REP = 32
    feat = (edge_feature + DEPTH_SIZE * (jnp.arange(E, dtype=jnp.int32) % REP)
            ).reshape(NW, NCHUNK, K)
    onehot = jnp.tile(jnp.eye(DEPTH_SIZE, D, dtype=jnp.float32), (REP, 1))

    (pc,) = _sc_agg(onehot, feat, agg)
    (px0,) = _sc_agg(data, src, agg)

    def edge_mat(W_edge, b_msg, b_edge):
        m = jnp.dot(emb, W_edge) + (b_msg + b_edge)[None, :]
        return jnp.zeros((D, D), jnp.float32).at[:DEPTH_SIZE].set(m)

    x1 = _dense(px0, pc, data, W_msg0, edge_mat(W_edge0, b_msg0, b_edge0),
                gamma0.reshape(1, D), beta0.reshape(1, D))

    (px1,) = _sc_agg(x1, src, agg)

    out = _dense(px1, pc, x1, W_msg1, edge_mat(W_edge1, b_msg1, b_edge1),
                 gamma1.reshape(1, D), beta1.reshape(1, D))
    return out
